# Initial kernel scaffold; baseline (speedup 1.0000x reference)
import jax
import jax.numpy as jnp
from jax.experimental import pallas as pl


def kernel(fmp_1, fmp_2, fmp_3, params):
    raise NotImplementedError("write your pallas kernel here")


# speedup vs baseline: 1.2091x; 1.2091x over previous
import functools

import numpy as np
import jax
import jax.numpy as jnp
from jax.experimental import pallas as pl
from jax.experimental.pallas import tpu as pltpu

_CP = getattr(pltpu, "CompilerParams", None) or getattr(pltpu, "TPUCompilerParams")

_STRIDES = (8, 16, 32)
_NC = 20
_ANCH = np.array([[[10, 13, 12], [16, 30, 20], [33, 23, 28]],
                  [[30, 61, 40], [62, 45, 50], [59, 119, 70]],
                  [[116, 90, 100], [156, 198, 150], [373, 326, 300]]],
                 np.float32)
_SCALE = 160.0


def _rup(x, m):
    return (x + m - 1) // m * m


def _vmem(n):
    return _CP(dimension_semantics=("parallel",) * n,
               vmem_limit_bytes=100 * 1024 * 1024)


def _mm(x, w, b=None, act=False):
    B, M, K = x.shape
    N = w.shape[1]
    Mp = _rup(M, 8)
    if Mp != M:
        x = jnp.pad(x, ((0, 0), (0, Mp - M), (0, 0)))
    if b is None:
        b = jnp.zeros((N,), jnp.float32)

    def kern(x_ref, w_ref, b_ref, o_ref):
        acc = jnp.dot(x_ref[0], w_ref[...],
                      preferred_element_type=jnp.float32)
        acc = acc + b_ref[...]
        if act:
            acc = jnp.maximum(acc, 0.1 * acc)
        o_ref[0] = acc

    out = pl.pallas_call(
        kern,
        grid=(B,),
        in_specs=[pl.BlockSpec((1, Mp, K), lambda i: (i, 0, 0)),
                  pl.BlockSpec((K, N), lambda i: (0, 0)),
                  pl.BlockSpec((1, N), lambda i: (0, 0))],
        out_specs=pl.BlockSpec((1, Mp, N), lambda i: (i, 0, 0)),
        out_shape=jax.ShapeDtypeStruct((B, Mp, N), jnp.float32),
        compiler_params=_vmem(1),
    )(x, w, b.reshape(1, N))
    return out[:, :M, :]


def _conv3(x, w, b):
    B, S0, S1, S2, C = x.shape
    N = w.shape[-1]
    Dp = _rup(S2 + 2, 8)
    Wp = S1 + 2
    Sh = Wp * Dp
    M = S0 * Sh
    M2 = M + 2 * Sh + 2 * Dp
    Vtot = M2 + 8
    xp = jnp.pad(x, ((0, 0), (1, 1), (1, 1), (1, Dp - S2 - 1), (0, 0)))
    xp = xp.reshape(B, (S0 + 2) * Sh, C)
    xp = jnp.pad(xp, ((0, 0), (0, Vtot - (S0 + 2) * Sh), (0, 0)))

    nb = 1
    while (27 * C * (N // nb) * 4 > 20 * 1024 * 1024) or (N % nb):
        nb += 1
    Nb = N // nb

    per = max(1, (512 * 1024 // (C * 4)) // Sh)
    while S0 % per:
        per -= 1
    CH = per * Sh

    def kern(x_ref, w_ref, b_ref, o_ref, sc_ref):
        for k in (1, 2):
            sc_ref[k - 1] = x_ref[0, k:k + M2, :]
        for c0 in range(0, S0, per):
            base = c0 * Sh
            acc = None
            for i in range(3):
                for j in range(3):
                    for k in range(3):
                        off = base + i * Sh + j * Dp
                        t = (i * 3 + j) * 3 + k
                        if k == 0:
                            lhs = x_ref[0, off:off + CH, :]
                        else:
                            lhs = sc_ref[k - 1, off:off + CH, :]
                        d = jnp.dot(lhs, w_ref[t],
                                    preferred_element_type=jnp.float32)
                        acc = d if acc is None else acc + d
            acc = acc + b_ref[...]
            o_ref[0, base:base + CH, :] = jnp.maximum(acc, 0.1 * acc)

    out = pl.pallas_call(
        kern,
        grid=(B, nb),
        in_specs=[pl.BlockSpec((1, Vtot, C), lambda i, n: (i, 0, 0)),
                  pl.BlockSpec((27, C, Nb), lambda i, n: (0, 0, n)),
                  pl.BlockSpec((1, Nb), lambda i, n: (0, n))],
        out_specs=pl.BlockSpec((1, M, Nb), lambda i, n: (i, 0, n)),
        out_shape=jax.ShapeDtypeStruct((B, M, N), jnp.float32),
        scratch_shapes=[pltpu.VMEM((2, M2, C), jnp.float32)],
        compiler_params=_vmem(2),
    )(xp, w, b.reshape(1, N))
    out = out.reshape(B, S0, Wp, Dp, N)[:, :, :S1, :S2, :]
    return out


def _interp_mat(n_out, n_in):
    pos = np.arange(n_out) * (n_in - 1) / max(n_out - 1, 1)
    i0 = np.floor(pos).astype(np.int64)
    i1 = np.minimum(i0 + 1, n_in - 1)
    fw = pos - i0
    m = np.zeros((n_out, n_in), np.float32)
    m[np.arange(n_out), i0] += 1.0 - fw
    m[np.arange(n_out), i1] += fw
    return m


@functools.lru_cache(None)
def _upsample_mat(s0, s1, s2):
    u = np.kron(np.kron(_interp_mat(2 * s0, s0), _interp_mat(2 * s1, s1)),
                _interp_mat(2 * s2, s2))
    return u


def _upsample(x, s):
    B, V, C = x.shape
    Vo = 8 * V
    Vp = _rup(V, 128)
    u = np.zeros((Vo, Vp), np.float32)
    u[:, :V] = _upsample_mat(s, s, s)
    u = jnp.asarray(u)
    x2 = jnp.pad(x, ((0, 0), (0, Vp - V), (0, 0)))
    x2 = x2.transpose(1, 0, 2).reshape(Vp, B * C)
    nb = next(n for n in (4, 5, 2, 1) if Vo % n == 0 and (Vo // n) % 8 == 0)
    Mb = Vo // nb

    def kern(u_ref, x_ref, o_ref):
        o_ref[...] = jnp.dot(u_ref[...], x_ref[...],
                             preferred_element_type=jnp.float32)

    out = pl.pallas_call(
        kern,
        grid=(nb,),
        in_specs=[pl.BlockSpec((Mb, Vp), lambda m: (m, 0)),
                  pl.BlockSpec((Vp, B * C), lambda m: (0, 0))],
        out_specs=pl.BlockSpec((Mb, B * C), lambda m: (m, 0)),
        out_shape=jax.ShapeDtypeStruct((Vo, B * C), jnp.float32),
        compiler_params=_vmem(1),
    )(u, x2)
    return out.reshape(Vo, B, C).transpose(1, 0, 2)


def _decode(pred, sdim, stride, anch):
    B, V, _ = pred.shape
    Vp = _rup(V, 128)
    pt = jnp.pad(pred, ((0, 0), (0, Vp - V), (0, 0))).transpose(0, 2, 1)
    W = D = sdim

    def kern(p_ref, ob_ref, os_ref):
        lane = jax.lax.broadcasted_iota(jnp.int32, (1, Vp), 1)
        gx = ((lane // D) % W).astype(jnp.float32)
        gy = (lane // (W * D)).astype(jnp.float32)
        gz = (lane % D).astype(jnp.float32)
        g = (gx, gy, gz)
        for a in range(3):
            c = 27 * a
            conf = jax.nn.sigmoid(p_ref[0, c:c + 1, :])
            lg = p_ref[0, c + 7:c + 27, :]
            mx = jnp.max(lg, axis=0, keepdims=True)
            e = jnp.exp(lg - mx)
            os_ref[0, 20 * a:20 * a + 20, :] = (
                conf * e / jnp.sum(e, axis=0, keepdims=True))
            for m in range(3):
                cm = (jax.nn.sigmoid(p_ref[0, c + 1 + m:c + 2 + m, :])
                      + g[m]) * stride
                half = (0.5 * float(anch[a, m])) * jnp.exp(
                    p_ref[0, c + 4 + m:c + 5 + m, :])
                ob_ref[0, 6 * a + m:6 * a + m + 1, :] = jnp.clip(
                    (cm - half) * (1.0 / _SCALE), 0.0, 1.0)
                ob_ref[0, 6 * a + 3 + m:6 * a + 4 + m, :] = jnp.clip(
                    (cm + half) * (1.0 / _SCALE), 0.0, 1.0)

    ob, os_ = pl.pallas_call(
        kern,
        grid=(B,),
        in_specs=[pl.BlockSpec((1, 81, Vp), lambda i: (i, 0, 0))],
        out_specs=[pl.BlockSpec((1, 24, Vp), lambda i: (i, 0, 0)),
                   pl.BlockSpec((1, 64, Vp), lambda i: (i, 0, 0))],
        out_shape=[jax.ShapeDtypeStruct((B, 24, Vp), jnp.float32),
                   jax.ShapeDtypeStruct((B, 64, Vp), jnp.float32)],
        compiler_params=_vmem(1),
    )(pt)
    boxes = ob[:, :18, :V].transpose(0, 2, 1).reshape(B, V * 3, 6)
    scores = os_[:, :60, :V].transpose(0, 2, 1).reshape(B, V * 3, _NC)
    return boxes, scores


def _cl(x):
    B, C = x.shape[0], x.shape[1]
    return x.transpose(0, 2, 3, 4, 1).reshape(B, -1, C)


def _w1(p):
    w = p['w'] * p['s'][:, None, None, None, None]
    return w.reshape(w.shape[0], w.shape[1]).T, p['b']


def _w3(p):
    w = p['w'] * p['s'][:, None, None, None, None]
    w = w.transpose(2, 3, 4, 1, 0).reshape(27, w.shape[1], w.shape[0])
    return w, p['b']


def _run_set(x, plist, sdim):
    B = x.shape[0]
    for idx, p in enumerate(plist):
        if p['w'].shape[-1] == 1:
            wk, bk = _w1(p)
            x = _mm(x, wk, bk, act=True)
        else:
            wk, bk = _w3(p)
            x = x.reshape(B, sdim, sdim, sdim, x.shape[-1])
            x = _conv3(x, wk, bk).reshape(B, sdim ** 3, -1)
    return x


def kernel(fmp_1, fmp_2, fmp_3, params):
    B = fmp_1.shape[0]
    x1, x2, x3 = _cl(fmp_1), _cl(fmp_2), _cl(fmp_3)

    f3 = _run_set(x3, params['conv_set_3'], 5)
    w, b = _w1(params['conv_1x1_3'])
    f3_up = _upsample(_mm(f3, w, b, act=True), 5)

    f2 = _run_set(jnp.concatenate([x2, f3_up], -1), params['conv_set_2'], 10)
    w, b = _w1(params['conv_1x1_2'])
    f2_up = _upsample(_mm(f2, w, b, act=True), 10)

    f1 = _run_set(jnp.concatenate([x1, f2_up], -1), params['conv_set_1'], 20)

    preds = []
    for f, sdim, ek, pk in ((f1, 20, 'extra_conv_1', 'pred_1'),
                            (f2, 10, 'extra_conv_2', 'pred_2'),
                            (f3, 5, 'extra_conv_3', 'pred_3')):
        we, be = _w3(params[ek])
        h = _conv3(f.reshape(B, sdim, sdim, sdim, -1), we, be)
        h = h.reshape(B, sdim ** 3, -1)
        wp = params[pk]['w']
        wp = wp.reshape(wp.shape[0], wp.shape[1]).T
        preds.append(_mm(h, wp, params[pk]['b'], act=False))

    outs = [_decode(preds[i], (20, 10, 5)[i], float(_STRIDES[i]), _ANCH[i])
            for i in range(3)]
    boxes = jnp.concatenate([o[0] for o in outs], 1)
    scores = jnp.concatenate([o[1] for o in outs], 1)
    return boxes, scores


# speedup vs baseline: 1.6301x; 1.3482x over previous
import functools

import numpy as np
import jax
import jax.numpy as jnp
from jax.experimental import pallas as pl
from jax.experimental.pallas import tpu as pltpu

_CP = getattr(pltpu, "CompilerParams", None) or getattr(pltpu, "TPUCompilerParams")

_STRIDES = (8, 16, 32)
_NC = 20
_ANCH = np.array([[[10, 13, 12], [16, 30, 20], [33, 23, 28]],
                  [[30, 61, 40], [62, 45, 50], [59, 119, 70]],
                  [[116, 90, 100], [156, 198, 150], [373, 326, 300]]],
                 np.float32)
_SCALE = 160.0


def _rup(x, m):
    return (x + m - 1) // m * m


def _vmem(n):
    return _CP(dimension_semantics=("parallel",) * n,
               vmem_limit_bytes=100 * 1024 * 1024)


def _mm(x, w, b=None, act=False):
    B, Mp, K = x.shape
    N = w.shape[1]
    if b is None:
        b = jnp.zeros((N,), jnp.float32)

    def kern(x_ref, w_ref, b_ref, o_ref):
        acc = jnp.dot(x_ref[0], w_ref[...],
                      preferred_element_type=jnp.float32)
        acc = acc + b_ref[...]
        if act:
            acc = jnp.maximum(acc, 0.1 * acc)
        o_ref[0] = acc

    out = pl.pallas_call(
        kern,
        grid=(B,),
        in_specs=[pl.BlockSpec((1, Mp, K), lambda i: (i, 0, 0)),
                  pl.BlockSpec((K, N), lambda i: (0, 0)),
                  pl.BlockSpec((1, N), lambda i: (0, 0))],
        out_specs=pl.BlockSpec((1, Mp, N), lambda i: (i, 0, 0)),
        out_shape=jax.ShapeDtypeStruct((B, Mp, N), jnp.float32),
        compiler_params=_vmem(1),
    )(x, w, b.reshape(1, N))
    return out


def _conv3(x, w, b, dims):
    B, Vp, C = x.shape
    S0, S1, S2 = dims
    N = w.shape[-1]
    Dp = _rup(S2 + 2, 8)
    Sh = (S1 + 2) * Dp
    M2 = S0 * Sh + 2 * Sh + 2 * Dp

    nb = 1
    while (27 * C * (N // nb) * 4 > 20 * 1024 * 1024) or (N % nb):
        nb += 1
    Nb = N // nb

    per = max(1, (512 * 1024 // (C * 4)) // Sh)
    while S0 % per:
        per -= 1
    CH = per * Sh

    def kern(x_ref, w_ref, b_ref, o_ref, sc_ref):
        @pl.when(pl.program_id(1) == 0)
        def _embed():
            sc_ref[1] = jnp.zeros((M2, C), jnp.float32)
            for h in range(S0):
                for wi in range(S1):
                    dst = (h + 1) * Sh + (wi + 1) * Dp
                    src = (h * S1 + wi) * S2
                    sc_ref[1, dst:dst + S2, :] = x_ref[0, src:src + S2, :]
            sc_ref[0, 0:8, :] = jnp.zeros((8, C), jnp.float32)
            sc_ref[0, 1:M2, :] = sc_ref[1, 0:M2 - 1, :]
            sc_ref[2, M2 - 8:M2, :] = jnp.zeros((8, C), jnp.float32)
            sc_ref[2, 0:M2 - 1, :] = sc_ref[1, 1:M2, :]

        for c0 in range(0, S0, per):
            base = c0 * Sh
            acc = None
            for i in range(3):
                for j in range(3):
                    for k in range(3):
                        off = base + i * Sh + j * Dp
                        t = (i * 3 + j) * 3 + k
                        lhs = sc_ref[k, off:off + CH, :]
                        d = jnp.dot(lhs, w_ref[t],
                                    preferred_element_type=jnp.float32)
                        acc = d if acc is None else acc + d
            acc = acc + b_ref[...]
            acc = jnp.maximum(acc, 0.1 * acc)
            if c0 == 0 and Vp > S0 * S1 * S2:
                o_ref[0, Vp - 8:Vp, :] = jnp.zeros((8, Nb), jnp.float32)
            for h in range(c0, c0 + per):
                for wi in range(S1):
                    src = (h - c0) * Sh + wi * Dp
                    dst = (h * S1 + wi) * S2
                    o_ref[0, dst:dst + S2, :] = acc[src:src + S2, :]

    out = pl.pallas_call(
        kern,
        grid=(B, nb),
        in_specs=[pl.BlockSpec((1, Vp, C), lambda i, n: (i, 0, 0)),
                  pl.BlockSpec((27, C, Nb), lambda i, n: (0, 0, n)),
                  pl.BlockSpec((1, Nb), lambda i, n: (0, n))],
        out_specs=pl.BlockSpec((1, Vp, Nb), lambda i, n: (i, 0, n)),
        out_shape=jax.ShapeDtypeStruct((B, Vp, N), jnp.float32),
        scratch_shapes=[pltpu.VMEM((3, M2, C), jnp.float32)],
        compiler_params=_CP(dimension_semantics=("parallel", "arbitrary"),
                            vmem_limit_bytes=100 * 1024 * 1024),
    )(x, w, b.reshape(1, N))
    return out


def _interp_mat(n_out, n_in):
    pos = np.arange(n_out) * (n_in - 1) / max(n_out - 1, 1)
    i0 = np.floor(pos).astype(np.int64)
    i1 = np.minimum(i0 + 1, n_in - 1)
    fw = pos - i0
    m = np.zeros((n_out, n_in), np.float32)
    m[np.arange(n_out), i0] += 1.0 - fw
    m[np.arange(n_out), i1] += fw
    return m


@functools.lru_cache(None)
def _upsample_mat(s0, s1, s2):
    u = np.kron(np.kron(_interp_mat(2 * s0, s0), _interp_mat(2 * s1, s1)),
                _interp_mat(2 * s2, s2))
    return u


def _upsample(x, s):
    B, Vin, C = x.shape
    V = s ** 3
    Vo = 8 * V
    Vp = _rup(Vin, 128)
    u = np.zeros((Vo, Vp), np.float32)
    u[:, :V] = _upsample_mat(s, s, s)
    u = jnp.asarray(u)
    x2 = jnp.pad(x, ((0, 0), (0, Vp - Vin), (0, 0)))
    x2 = x2.transpose(1, 0, 2).reshape(Vp, B * C)
    nb = next(n for n in (4, 5, 2, 1) if Vo % n == 0 and (Vo // n) % 8 == 0)
    Mb = Vo // nb

    def kern(u_ref, x_ref, o_ref):
        o_ref[...] = jnp.dot(u_ref[...], x_ref[...],
                             preferred_element_type=jnp.float32)

    out = pl.pallas_call(
        kern,
        grid=(nb,),
        in_specs=[pl.BlockSpec((Mb, Vp), lambda m: (m, 0)),
                  pl.BlockSpec((Vp, B * C), lambda m: (0, 0))],
        out_specs=pl.BlockSpec((Mb, B * C), lambda m: (m, 0)),
        out_shape=jax.ShapeDtypeStruct((Vo, B * C), jnp.float32),
        compiler_params=_vmem(1),
    )(u, x2)
    return out.reshape(Vo, B, C).transpose(1, 0, 2)


def _decode(pred, sdim, stride, anch):
    B, Vin, _ = pred.shape
    V = sdim ** 3
    Vp = _rup(Vin, 128)
    pt = jnp.pad(pred, ((0, 0), (0, Vp - Vin), (0, 0))).transpose(0, 2, 1)
    W = D = sdim

    def kern(p_ref, ob_ref, os_ref):
        lane = jax.lax.broadcasted_iota(jnp.int32, (1, Vp), 1)
        gx = ((lane // D) % W).astype(jnp.float32)
        gy = (lane // (W * D)).astype(jnp.float32)
        gz = (lane % D).astype(jnp.float32)
        g = (gx, gy, gz)
        for a in range(3):
            c = 27 * a
            conf = jax.nn.sigmoid(p_ref[0, c:c + 1, :])
            lg = p_ref[0, c + 7:c + 27, :]
            mx = jnp.max(lg, axis=0, keepdims=True)
            e = jnp.exp(lg - mx)
            os_ref[0, 20 * a:20 * a + 20, :] = (
                conf * e / jnp.sum(e, axis=0, keepdims=True))
            for m in range(3):
                cm = (jax.nn.sigmoid(p_ref[0, c + 1 + m:c + 2 + m, :])
                      + g[m]) * stride
                half = (0.5 * float(anch[a, m])) * jnp.exp(
                    p_ref[0, c + 4 + m:c + 5 + m, :])
                ob_ref[0, 6 * a + m:6 * a + m + 1, :] = jnp.clip(
                    (cm - half) * (1.0 / _SCALE), 0.0, 1.0)
                ob_ref[0, 6 * a + 3 + m:6 * a + 4 + m, :] = jnp.clip(
                    (cm + half) * (1.0 / _SCALE), 0.0, 1.0)

    ob, os_ = pl.pallas_call(
        kern,
        grid=(B,),
        in_specs=[pl.BlockSpec((1, 81, Vp), lambda i: (i, 0, 0))],
        out_specs=[pl.BlockSpec((1, 24, Vp), lambda i: (i, 0, 0)),
                   pl.BlockSpec((1, 64, Vp), lambda i: (i, 0, 0))],
        out_shape=[jax.ShapeDtypeStruct((B, 24, Vp), jnp.float32),
                   jax.ShapeDtypeStruct((B, 64, Vp), jnp.float32)],
        compiler_params=_vmem(1),
    )(pt)
    boxes = ob[:, :18, :V].transpose(0, 2, 1).reshape(B, V * 3, 6)
    scores = os_[:, :60, :V].transpose(0, 2, 1).reshape(B, V * 3, _NC)
    return boxes, scores


def _cl(x):
    B, C = x.shape[0], x.shape[1]
    return x.transpose(0, 2, 3, 4, 1).reshape(B, -1, C)


def _w1(p):
    w = p['w'] * p['s'][:, None, None, None, None]
    return w.reshape(w.shape[0], w.shape[1]).T, p['b']


def _w3(p):
    w = p['w'] * p['s'][:, None, None, None, None]
    w = w.transpose(2, 3, 4, 1, 0).reshape(27, w.shape[1], w.shape[0])
    return w, p['b']


def _run_set(x, plist, sdim):
    for p in plist:
        if p['w'].shape[-1] == 1:
            wk, bk = _w1(p)
            x = _mm(x, wk, bk, act=True)
        else:
            wk, bk = _w3(p)
            x = _conv3(x, wk, bk, (sdim, sdim, sdim))
    return x


def kernel(fmp_1, fmp_2, fmp_3, params):
    B = fmp_1.shape[0]
    x1, x2, x3 = _cl(fmp_1), _cl(fmp_2), _cl(fmp_3)
    x3 = jnp.pad(x3, ((0, 0), (0, 3), (0, 0)))

    f3 = _run_set(x3, params['conv_set_3'], 5)
    w, b = _w1(params['conv_1x1_3'])
    f3_up = _upsample(_mm(f3, w, b, act=True), 5)

    f2 = _run_set(jnp.concatenate([x2, f3_up], -1), params['conv_set_2'], 10)
    w, b = _w1(params['conv_1x1_2'])
    f2_up = _upsample(_mm(f2, w, b, act=True), 10)

    f1 = _run_set(jnp.concatenate([x1, f2_up], -1), params['conv_set_1'], 20)

    preds = []
    for f, sdim, ek, pk in ((f1, 20, 'extra_conv_1', 'pred_1'),
                            (f2, 10, 'extra_conv_2', 'pred_2'),
                            (f3, 5, 'extra_conv_3', 'pred_3')):
        we, be = _w3(params[ek])
        h = _conv3(f, we, be, (sdim, sdim, sdim))
        wp = params[pk]['w']
        wp = wp.reshape(wp.shape[0], wp.shape[1]).T
        preds.append(_mm(h, wp, params[pk]['b'], act=False))

    outs = [_decode(preds[i], (20, 10, 5)[i], float(_STRIDES[i]), _ANCH[i])
            for i in range(3)]
    boxes = jnp.concatenate([o[0] for o in outs], 1)
    scores = jnp.concatenate([o[1] for o in outs], 1)
    return boxes, scores


# speedup vs baseline: 1.6799x; 1.0306x over previous
import functools

import numpy as np
import jax
import jax.numpy as jnp
from jax.experimental import pallas as pl
from jax.experimental.pallas import tpu as pltpu

_CP = getattr(pltpu, "CompilerParams", None) or getattr(pltpu, "TPUCompilerParams")

_STRIDES = (8, 16, 32)
_NC = 20
_ANCH = np.array([[[10, 13, 12], [16, 30, 20], [33, 23, 28]],
                  [[30, 61, 40], [62, 45, 50], [59, 119, 70]],
                  [[116, 90, 100], [156, 198, 150], [373, 326, 300]]],
                 np.float32)
_SCALE = 160.0


def _rup(x, m):
    return (x + m - 1) // m * m


def _vmem(n):
    return _CP(dimension_semantics=("parallel",) * n,
               vmem_limit_bytes=100 * 1024 * 1024)


def _mm(x, w, b=None, act=False):
    B, Mp, K = x.shape
    N = w.shape[1]
    if b is None:
        b = jnp.zeros((N,), jnp.float32)

    def kern(x_ref, w_ref, b_ref, o_ref):
        acc = jnp.dot(x_ref[0], w_ref[...],
                      preferred_element_type=jnp.float32)
        acc = acc + b_ref[...]
        if act:
            acc = jnp.maximum(acc, 0.1 * acc)
        o_ref[0] = acc

    out = pl.pallas_call(
        kern,
        grid=(B,),
        in_specs=[pl.BlockSpec((1, Mp, K), lambda i: (i, 0, 0)),
                  pl.BlockSpec((K, N), lambda i: (0, 0)),
                  pl.BlockSpec((1, N), lambda i: (0, 0))],
        out_specs=pl.BlockSpec((1, Mp, N), lambda i: (i, 0, 0)),
        out_shape=jax.ShapeDtypeStruct((B, Mp, N), jnp.float32),
        compiler_params=_vmem(1),
    )(x, w, b.reshape(1, N))
    return out


def _conv3(x, w, b, dims):
    B, Vp, C = x.shape
    S0, S1, S2 = dims
    N = w.shape[-1]
    Dp = _rup(S2 + 2, 8)
    Sh = (S1 + 2) * Dp
    M2 = S0 * Sh + 2 * Sh + 2 * Dp

    nb = 1
    while (27 * C * (N // nb) * 4 > 20 * 1024 * 1024) or (N % nb):
        nb += 1
    Nb = N // nb

    per = max(1, (512 * 1024 // (C * 4)) // Sh)
    while S0 % per:
        per -= 1
    CH = per * Sh

    def kern(x_ref, w_ref, b_ref, o_ref, sc_ref):
        @pl.when(pl.program_id(1) == 0)
        def _embed():
            sc_ref[1] = jnp.zeros((M2, C), jnp.float32)
            for h in range(S0):
                for wi in range(S1):
                    dst = (h + 1) * Sh + (wi + 1) * Dp
                    src = (h * S1 + wi) * S2
                    sc_ref[1, dst:dst + S2, :] = x_ref[0, src:src + S2, :]
            sc_ref[0, 0:8, :] = jnp.zeros((8, C), jnp.float32)
            sc_ref[0, 1:M2, :] = sc_ref[1, 0:M2 - 1, :]
            sc_ref[2, M2 - 8:M2, :] = jnp.zeros((8, C), jnp.float32)
            sc_ref[2, 0:M2 - 1, :] = sc_ref[1, 1:M2, :]

        for c0 in range(0, S0, per):
            base = c0 * Sh
            acc = None
            for i in range(3):
                for j in range(3):
                    for k in range(3):
                        off = base + i * Sh + j * Dp
                        t = (i * 3 + j) * 3 + k
                        lhs = sc_ref[k, off:off + CH, :]
                        d = jnp.dot(lhs, w_ref[t],
                                    preferred_element_type=jnp.float32)
                        acc = d if acc is None else acc + d
            acc = acc + b_ref[...]
            acc = jnp.maximum(acc, 0.1 * acc)
            if c0 == 0 and Vp > S0 * S1 * S2:
                o_ref[0, Vp - 8:Vp, :] = jnp.zeros((8, Nb), jnp.float32)
            for h in range(c0, c0 + per):
                for wi in range(S1):
                    src = (h - c0) * Sh + wi * Dp
                    dst = (h * S1 + wi) * S2
                    o_ref[0, dst:dst + S2, :] = acc[src:src + S2, :]

    out = pl.pallas_call(
        kern,
        grid=(B, nb),
        in_specs=[pl.BlockSpec((1, Vp, C), lambda i, n: (i, 0, 0)),
                  pl.BlockSpec((27, C, Nb), lambda i, n: (0, 0, n)),
                  pl.BlockSpec((1, Nb), lambda i, n: (0, n))],
        out_specs=pl.BlockSpec((1, Vp, Nb), lambda i, n: (i, 0, n)),
        out_shape=jax.ShapeDtypeStruct((B, Vp, N), jnp.float32),
        scratch_shapes=[pltpu.VMEM((3, M2, C), jnp.float32)],
        compiler_params=_CP(dimension_semantics=("parallel", "arbitrary"),
                            vmem_limit_bytes=100 * 1024 * 1024),
    )(x, w, b.reshape(1, N))
    return out


def _set_first(x_nat, up, w, b):
    B, C1, V = x_nat.shape
    Vp = _rup(V, 8)
    N = w.shape[1]
    w1 = w[:C1]
    w2 = w[C1:] if w.shape[0] > C1 else None
    two = up is not None

    def kern(*refs):
        if two:
            x1_ref, x2_ref, wa_ref, wb_ref, b_ref, o_ref = refs
        else:
            x1_ref, wa_ref, b_ref, o_ref = refs
        acc = jax.lax.dot_general(
            x1_ref[0], wa_ref[...], (((0,), (0,)), ((), ())),
            preferred_element_type=jnp.float32)
        if two:
            acc = acc + jnp.dot(x2_ref[0, :V, :], wb_ref[...],
                                preferred_element_type=jnp.float32)
        acc = acc + b_ref[...]
        o_ref[0, :V, :] = jnp.maximum(acc, 0.1 * acc)

    ins = [x_nat]
    specs = [pl.BlockSpec((1, C1, V), lambda i: (i, 0, 0))]
    if two:
        ins.append(up)
        specs.append(pl.BlockSpec((1, up.shape[1], up.shape[2]),
                                  lambda i: (i, 0, 0)))
    ins.append(w1)
    specs.append(pl.BlockSpec(w1.shape, lambda i: (0, 0)))
    if two:
        ins.append(w2)
        specs.append(pl.BlockSpec(w2.shape, lambda i: (0, 0)))
    ins.append(b.reshape(1, N))
    specs.append(pl.BlockSpec((1, N), lambda i: (0, 0)))
    return pl.pallas_call(
        kern,
        grid=(B,),
        in_specs=specs,
        out_specs=pl.BlockSpec((1, Vp, N), lambda i: (i, 0, 0)),
        out_shape=jax.ShapeDtypeStruct((B, Vp, N), jnp.float32),
        compiler_params=_vmem(1),
    )(*ins)


def _interp_mat(n_out, n_in):
    pos = np.arange(n_out) * (n_in - 1) / max(n_out - 1, 1)
    i0 = np.floor(pos).astype(np.int64)
    i1 = np.minimum(i0 + 1, n_in - 1)
    fw = pos - i0
    m = np.zeros((n_out, n_in), np.float32)
    m[np.arange(n_out), i0] += 1.0 - fw
    m[np.arange(n_out), i1] += fw
    return m


def _upsample(x, s):
    B, Vin, C = x.shape
    s2 = s * s
    u = jnp.asarray(np.kron(_interp_mat(2 * s, s), _interp_mat(2 * s, s)))
    mh = _interp_mat(2 * s, s)

    def kern(x_ref, u_ref, o_ref, sc_ref):
        for h in range(s):
            xs = x_ref[0, h * s2:(h + 1) * s2, :]
            sc_ref[h] = jnp.dot(u_ref[...], xs,
                                preferred_element_type=jnp.float32)
        for oh in range(2 * s):
            nz = np.nonzero(mh[oh])[0]
            val = float(mh[oh, nz[0]]) * sc_ref[nz[0]]
            for i in nz[1:]:
                val = val + float(mh[oh, i]) * sc_ref[i]
            o_ref[0, oh * 4 * s2:(oh + 1) * 4 * s2, :] = val

    return pl.pallas_call(
        kern,
        grid=(B,),
        in_specs=[pl.BlockSpec((1, Vin, C), lambda i: (i, 0, 0)),
                  pl.BlockSpec((4 * s2, s2), lambda i: (0, 0))],
        out_specs=pl.BlockSpec((1, 8 * s * s2, C), lambda i: (i, 0, 0)),
        out_shape=jax.ShapeDtypeStruct((B, 8 * s * s2, C), jnp.float32),
        scratch_shapes=[pltpu.VMEM((s, 4 * s2, C), jnp.float32)],
        compiler_params=_vmem(1),
    )(x, u)


def _decode(pred, sdim, stride, anch):
    B, Vin, _ = pred.shape
    V = sdim ** 3
    Vp = _rup(Vin, 128)
    pt = jnp.pad(pred, ((0, 0), (0, Vp - Vin), (0, 0))).transpose(0, 2, 1)
    W = D = sdim

    def kern(p_ref, ob_ref, os_ref):
        lane = jax.lax.broadcasted_iota(jnp.int32, (1, Vp), 1)
        gx = ((lane // D) % W).astype(jnp.float32)
        gy = (lane // (W * D)).astype(jnp.float32)
        gz = (lane % D).astype(jnp.float32)
        g = (gx, gy, gz)
        for a in range(3):
            c = 27 * a
            conf = jax.nn.sigmoid(p_ref[0, c:c + 1, :])
            lg = p_ref[0, c + 7:c + 27, :]
            mx = jnp.max(lg, axis=0, keepdims=True)
            e = jnp.exp(lg - mx)
            os_ref[0, 20 * a:20 * a + 20, :] = (
                conf * e / jnp.sum(e, axis=0, keepdims=True))
            for m in range(3):
                cm = (jax.nn.sigmoid(p_ref[0, c + 1 + m:c + 2 + m, :])
                      + g[m]) * stride
                half = (0.5 * float(anch[a, m])) * jnp.exp(
                    p_ref[0, c + 4 + m:c + 5 + m, :])
                ob_ref[0, 6 * a + m:6 * a + m + 1, :] = jnp.clip(
                    (cm - half) * (1.0 / _SCALE), 0.0, 1.0)
                ob_ref[0, 6 * a + 3 + m:6 * a + 4 + m, :] = jnp.clip(
                    (cm + half) * (1.0 / _SCALE), 0.0, 1.0)

    ob, os_ = pl.pallas_call(
        kern,
        grid=(B,),
        in_specs=[pl.BlockSpec((1, 81, Vp), lambda i: (i, 0, 0))],
        out_specs=[pl.BlockSpec((1, 24, Vp), lambda i: (i, 0, 0)),
                   pl.BlockSpec((1, 64, Vp), lambda i: (i, 0, 0))],
        out_shape=[jax.ShapeDtypeStruct((B, 24, Vp), jnp.float32),
                   jax.ShapeDtypeStruct((B, 64, Vp), jnp.float32)],
        compiler_params=_vmem(1),
    )(pt)
    boxes = ob[:, :18, :V].transpose(0, 2, 1).reshape(B, V * 3, 6)
    scores = os_[:, :60, :V].transpose(0, 2, 1).reshape(B, V * 3, _NC)
    return boxes, scores


def _cl(x):
    B, C = x.shape[0], x.shape[1]
    return x.transpose(0, 2, 3, 4, 1).reshape(B, -1, C)


def _w1(p):
    w = p['w'] * p['s'][:, None, None, None, None]
    return w.reshape(w.shape[0], w.shape[1]).T, p['b']


def _w3(p):
    w = p['w'] * p['s'][:, None, None, None, None]
    w = w.transpose(2, 3, 4, 1, 0).reshape(27, w.shape[1], w.shape[0])
    return w, p['b']


def _run_set(x, plist, sdim):
    for p in plist:
        if p['w'].shape[-1] == 1:
            wk, bk = _w1(p)
            x = _mm(x, wk, bk, act=True)
        else:
            wk, bk = _w3(p)
            x = _conv3(x, wk, bk, (sdim, sdim, sdim))
    return x


def kernel(fmp_1, fmp_2, fmp_3, params):
    B = fmp_1.shape[0]
    xn1 = fmp_1.reshape(B, fmp_1.shape[1], -1)
    xn2 = fmp_2.reshape(B, fmp_2.shape[1], -1)
    xn3 = fmp_3.reshape(B, fmp_3.shape[1], -1)

    w, b = _w1(params['conv_set_3'][0])
    f3 = _run_set(_set_first(xn3, None, w, b), params['conv_set_3'][1:], 5)
    w, b = _w1(params['conv_1x1_3'])
    f3_up = _upsample(_mm(f3, w, b, act=True), 5)

    w, b = _w1(params['conv_set_2'][0])
    f2 = _run_set(_set_first(xn2, f3_up, w, b),
                  params['conv_set_2'][1:], 10)
    w, b = _w1(params['conv_1x1_2'])
    f2_up = _upsample(_mm(f2, w, b, act=True), 10)

    w, b = _w1(params['conv_set_1'][0])
    f1 = _run_set(_set_first(xn1, f2_up, w, b),
                  params['conv_set_1'][1:], 20)

    preds = []
    for f, sdim, ek, pk in ((f1, 20, 'extra_conv_1', 'pred_1'),
                            (f2, 10, 'extra_conv_2', 'pred_2'),
                            (f3, 5, 'extra_conv_3', 'pred_3')):
        we, be = _w3(params[ek])
        h = _conv3(f, we, be, (sdim, sdim, sdim))
        wp = params[pk]['w']
        wp = wp.reshape(wp.shape[0], wp.shape[1]).T
        preds.append(_mm(h, wp, params[pk]['b'], act=False))

    outs = [_decode(preds[i], (20, 10, 5)[i], float(_STRIDES[i]), _ANCH[i])
            for i in range(3)]
    boxes = jnp.concatenate([o[0] for o in outs], 1)
    scores = jnp.concatenate([o[1] for o in outs], 1)
    return boxes, scores


# speedup vs baseline: 1.7244x; 1.0264x over previous
import functools

import numpy as np
import jax
import jax.numpy as jnp
from jax.experimental import pallas as pl
from jax.experimental.pallas import tpu as pltpu

_CP = getattr(pltpu, "CompilerParams", None) or getattr(pltpu, "TPUCompilerParams")

_STRIDES = (8, 16, 32)
_NC = 20
_ANCH = np.array([[[10, 13, 12], [16, 30, 20], [33, 23, 28]],
                  [[30, 61, 40], [62, 45, 50], [59, 119, 70]],
                  [[116, 90, 100], [156, 198, 150], [373, 326, 300]]],
                 np.float32)
_SCALE = 160.0


def _rup(x, m):
    return (x + m - 1) // m * m


def _vmem(n):
    return _CP(dimension_semantics=("parallel",) * n,
               vmem_limit_bytes=100 * 1024 * 1024)


def _mm(x, w, b=None, act=False):
    B, Mp, K = x.shape
    N = w.shape[1]
    if b is None:
        b = jnp.zeros((N,), jnp.float32)

    def kern(x_ref, w_ref, b_ref, o_ref):
        acc = jnp.dot(x_ref[0], w_ref[...],
                      preferred_element_type=jnp.float32)
        acc = acc + b_ref[...]
        if act:
            acc = jnp.maximum(acc, 0.1 * acc)
        o_ref[0] = acc

    out = pl.pallas_call(
        kern,
        grid=(B,),
        in_specs=[pl.BlockSpec((1, Mp, K), lambda i: (i, 0, 0)),
                  pl.BlockSpec((K, N), lambda i: (0, 0)),
                  pl.BlockSpec((1, N), lambda i: (0, 0))],
        out_specs=pl.BlockSpec((1, Mp, N), lambda i: (i, 0, 0)),
        out_shape=jax.ShapeDtypeStruct((B, Mp, N), jnp.float32),
        compiler_params=_vmem(1),
    )(x, w, b.reshape(1, N))
    return out


def _conv3(x, w, b, dims, pre=None):
    B, Vp, C0 = x.shape
    S0, S1, S2 = dims
    C = w.shape[1]
    N = w.shape[-1]
    Dp = _rup(S2 + 2, 8)
    Sh = (S1 + 2) * Dp
    M2 = S0 * Sh + 2 * Sh + 2 * Dp

    nb = 1
    while (27 * C * (N // nb) * 4 > 20 * 1024 * 1024) or (N % nb):
        nb += 1
    Nb = N // nb

    per = max(1, (512 * 1024 // (C * 4)) // Sh)
    while S0 % per:
        per -= 1
    CH = per * Sh

    def kern(*refs):
        if pre is None:
            x_ref, w_ref, b_ref, o_ref, sc_ref = refs
        else:
            x_ref, wp_ref, bp_ref, w_ref, b_ref, o_ref, sc_ref = refs

        @pl.when(pl.program_id(1) == 0)
        def _embed():
            sc_ref[1] = jnp.zeros((M2, C), jnp.float32)
            for h in range(S0):
                src0 = h * S1 * S2
                if pre is None:
                    ysl = x_ref[0, src0:src0 + S1 * S2, :]
                else:
                    ysl = jnp.dot(x_ref[0, src0:src0 + S1 * S2, :],
                                  wp_ref[...],
                                  preferred_element_type=jnp.float32)
                    ysl = ysl + bp_ref[...]
                    ysl = jnp.maximum(ysl, 0.1 * ysl)
                for wi in range(S1):
                    dst = (h + 1) * Sh + (wi + 1) * Dp
                    sc_ref[1, dst:dst + S2, :] = ysl[wi * S2:(wi + 1) * S2, :]
            sc_ref[0, 0:8, :] = jnp.zeros((8, C), jnp.float32)
            sc_ref[0, 1:M2, :] = sc_ref[1, 0:M2 - 1, :]
            sc_ref[2, M2 - 8:M2, :] = jnp.zeros((8, C), jnp.float32)
            sc_ref[2, 0:M2 - 1, :] = sc_ref[1, 1:M2, :]

        for c0 in range(0, S0, per):
            base = c0 * Sh
            acc = None
            for i in range(3):
                for j in range(3):
                    for k in range(3):
                        off = base + i * Sh + j * Dp
                        t = (i * 3 + j) * 3 + k
                        lhs = sc_ref[k, off:off + CH, :]
                        d = jnp.dot(lhs, w_ref[t],
                                    preferred_element_type=jnp.float32)
                        acc = d if acc is None else acc + d
            acc = acc + b_ref[...]
            acc = jnp.maximum(acc, 0.1 * acc)
            if c0 == 0 and Vp > S0 * S1 * S2:
                o_ref[0, Vp - 8:Vp, :] = jnp.zeros((8, Nb), jnp.float32)
            for h in range(c0, c0 + per):
                for wi in range(S1):
                    src = (h - c0) * Sh + wi * Dp
                    dst = (h * S1 + wi) * S2
                    o_ref[0, dst:dst + S2, :] = acc[src:src + S2, :]

    ins = [x]
    specs = [pl.BlockSpec((1, Vp, C0), lambda i, n: (i, 0, 0))]
    if pre is not None:
        ins += [pre[0], pre[1].reshape(1, C)]
        specs += [pl.BlockSpec((C0, C), lambda i, n: (0, 0)),
                  pl.BlockSpec((1, C), lambda i, n: (0, 0))]
    ins += [w, b.reshape(1, N)]
    specs += [pl.BlockSpec((27, C, Nb), lambda i, n: (0, 0, n)),
              pl.BlockSpec((1, Nb), lambda i, n: (0, n))]
    out = pl.pallas_call(
        kern,
        grid=(B, nb),
        in_specs=specs,
        out_specs=pl.BlockSpec((1, Vp, Nb), lambda i, n: (i, 0, n)),
        out_shape=jax.ShapeDtypeStruct((B, Vp, N), jnp.float32),
        scratch_shapes=[pltpu.VMEM((3, M2, C), jnp.float32)],
        compiler_params=_CP(dimension_semantics=("parallel", "arbitrary"),
                            vmem_limit_bytes=100 * 1024 * 1024),
    )(*ins)
    return out


def _set_first(x_nat, up, w, b):
    B, C1, V = x_nat.shape
    Vp = _rup(V, 8)
    N = w.shape[1]
    w1 = w[:C1]
    w2 = w[C1:] if w.shape[0] > C1 else None
    two = up is not None

    def kern(*refs):
        if two:
            x1_ref, x2_ref, wa_ref, wb_ref, b_ref, o_ref = refs
        else:
            x1_ref, wa_ref, b_ref, o_ref = refs
        acc = jax.lax.dot_general(
            x1_ref[0], wa_ref[...], (((0,), (0,)), ((), ())),
            preferred_element_type=jnp.float32)
        if two:
            acc = acc + jnp.dot(x2_ref[0, :V, :], wb_ref[...],
                                preferred_element_type=jnp.float32)
        acc = acc + b_ref[...]
        o_ref[0, :V, :] = jnp.maximum(acc, 0.1 * acc)

    ins = [x_nat]
    specs = [pl.BlockSpec((1, C1, V), lambda i: (i, 0, 0))]
    if two:
        ins.append(up)
        specs.append(pl.BlockSpec((1, up.shape[1], up.shape[2]),
                                  lambda i: (i, 0, 0)))
    ins.append(w1)
    specs.append(pl.BlockSpec(w1.shape, lambda i: (0, 0)))
    if two:
        ins.append(w2)
        specs.append(pl.BlockSpec(w2.shape, lambda i: (0, 0)))
    ins.append(b.reshape(1, N))
    specs.append(pl.BlockSpec((1, N), lambda i: (0, 0)))
    return pl.pallas_call(
        kern,
        grid=(B,),
        in_specs=specs,
        out_specs=pl.BlockSpec((1, Vp, N), lambda i: (i, 0, 0)),
        out_shape=jax.ShapeDtypeStruct((B, Vp, N), jnp.float32),
        compiler_params=_vmem(1),
    )(*ins)


def _interp_mat(n_out, n_in):
    pos = np.arange(n_out) * (n_in - 1) / max(n_out - 1, 1)
    i0 = np.floor(pos).astype(np.int64)
    i1 = np.minimum(i0 + 1, n_in - 1)
    fw = pos - i0
    m = np.zeros((n_out, n_in), np.float32)
    m[np.arange(n_out), i0] += 1.0 - fw
    m[np.arange(n_out), i1] += fw
    return m


def _upsample(x, s):
    B, Vin, C = x.shape
    s2 = s * s
    u = jnp.asarray(np.kron(_interp_mat(2 * s, s), _interp_mat(2 * s, s)))
    mh = _interp_mat(2 * s, s)

    def kern(x_ref, u_ref, o_ref, sc_ref):
        for h in range(s):
            xs = x_ref[0, h * s2:(h + 1) * s2, :]
            sc_ref[h] = jnp.dot(u_ref[...], xs,
                                preferred_element_type=jnp.float32)
        for oh in range(2 * s):
            nz = np.nonzero(mh[oh])[0]
            val = float(mh[oh, nz[0]]) * sc_ref[nz[0]]
            for i in nz[1:]:
                val = val + float(mh[oh, i]) * sc_ref[i]
            o_ref[0, oh * 4 * s2:(oh + 1) * 4 * s2, :] = val

    return pl.pallas_call(
        kern,
        grid=(B,),
        in_specs=[pl.BlockSpec((1, Vin, C), lambda i: (i, 0, 0)),
                  pl.BlockSpec((4 * s2, s2), lambda i: (0, 0))],
        out_specs=pl.BlockSpec((1, 8 * s * s2, C), lambda i: (i, 0, 0)),
        out_shape=jax.ShapeDtypeStruct((B, 8 * s * s2, C), jnp.float32),
        scratch_shapes=[pltpu.VMEM((s, 4 * s2, C), jnp.float32)],
        compiler_params=_vmem(1),
    )(x, u)


def _decode(pred, sdim, stride, anch):
    B, Vin, _ = pred.shape
    V = sdim ** 3
    Vp = _rup(Vin, 128)
    pt = jnp.pad(pred, ((0, 0), (0, Vp - Vin), (0, 0))).transpose(0, 2, 1)
    W = D = sdim

    def kern(p_ref, ob_ref, os_ref):
        lane = jax.lax.broadcasted_iota(jnp.int32, (1, Vp), 1)
        gx = ((lane // D) % W).astype(jnp.float32)
        gy = (lane // (W * D)).astype(jnp.float32)
        gz = (lane % D).astype(jnp.float32)
        g = (gx, gy, gz)
        for a in range(3):
            c = 27 * a
            conf = jax.nn.sigmoid(p_ref[0, c:c + 1, :])
            lg = p_ref[0, c + 7:c + 27, :]
            mx = jnp.max(lg, axis=0, keepdims=True)
            e = jnp.exp(lg - mx)
            os_ref[0, 20 * a:20 * a + 20, :] = (
                conf * e / jnp.sum(e, axis=0, keepdims=True))
            for m in range(3):
                cm = (jax.nn.sigmoid(p_ref[0, c + 1 + m:c + 2 + m, :])
                      + g[m]) * stride
                half = (0.5 * float(anch[a, m])) * jnp.exp(
                    p_ref[0, c + 4 + m:c + 5 + m, :])
                ob_ref[0, 6 * a + m:6 * a + m + 1, :] = jnp.clip(
                    (cm - half) * (1.0 / _SCALE), 0.0, 1.0)
                ob_ref[0, 6 * a + 3 + m:6 * a + 4 + m, :] = jnp.clip(
                    (cm + half) * (1.0 / _SCALE), 0.0, 1.0)

    ob, os_ = pl.pallas_call(
        kern,
        grid=(B,),
        in_specs=[pl.BlockSpec((1, 81, Vp), lambda i: (i, 0, 0))],
        out_specs=[pl.BlockSpec((1, 24, Vp), lambda i: (i, 0, 0)),
                   pl.BlockSpec((1, 64, Vp), lambda i: (i, 0, 0))],
        out_shape=[jax.ShapeDtypeStruct((B, 24, Vp), jnp.float32),
                   jax.ShapeDtypeStruct((B, 64, Vp), jnp.float32)],
        compiler_params=_vmem(1),
    )(pt)
    boxes = ob[:, :18, :V].transpose(0, 2, 1).reshape(B, V * 3, 6)
    scores = os_[:, :60, :V].transpose(0, 2, 1).reshape(B, V * 3, _NC)
    return boxes, scores


def _cl(x):
    B, C = x.shape[0], x.shape[1]
    return x.transpose(0, 2, 3, 4, 1).reshape(B, -1, C)


def _w1(p):
    w = p['w'] * p['s'][:, None, None, None, None]
    return w.reshape(w.shape[0], w.shape[1]).T, p['b']


def _w3(p):
    w = p['w'] * p['s'][:, None, None, None, None]
    w = w.transpose(2, 3, 4, 1, 0).reshape(27, w.shape[1], w.shape[0])
    return w, p['b']


def _run_set(x, plist, sdim):
    i = 0
    while i < len(plist):
        p = plist[i]
        if p['w'].shape[-1] == 1:
            if i + 1 < len(plist) and plist[i + 1]['w'].shape[-1] == 3:
                wk, bk = _w3(plist[i + 1])
                x = _conv3(x, wk, bk, (sdim,) * 3, pre=_w1(p))
                i += 2
                continue
            wk, bk = _w1(p)
            x = _mm(x, wk, bk, act=True)
        else:
            wk, bk = _w3(p)
            x = _conv3(x, wk, bk, (sdim,) * 3)
        i += 1
    return x


def kernel(fmp_1, fmp_2, fmp_3, params):
    B = fmp_1.shape[0]
    xn1 = fmp_1.reshape(B, fmp_1.shape[1], -1)
    xn2 = fmp_2.reshape(B, fmp_2.shape[1], -1)
    xn3 = fmp_3.reshape(B, fmp_3.shape[1], -1)

    w, b = _w1(params['conv_set_3'][0])
    f3 = _run_set(_set_first(xn3, None, w, b), params['conv_set_3'][1:], 5)
    w, b = _w1(params['conv_1x1_3'])
    f3_up = _upsample(_mm(f3, w, b, act=True), 5)

    w, b = _w1(params['conv_set_2'][0])
    f2 = _run_set(_set_first(xn2, f3_up, w, b),
                  params['conv_set_2'][1:], 10)
    w, b = _w1(params['conv_1x1_2'])
    f2_up = _upsample(_mm(f2, w, b, act=True), 10)

    w, b = _w1(params['conv_set_1'][0])
    f1mid = _run_set(_set_first(xn1, f2_up, w, b),
                     params['conv_set_1'][1:-1], 20)

    preds = []
    for f, sdim, ek, pk, pre in (
            (f1mid, 20, 'extra_conv_1', 'pred_1',
             _w1(params['conv_set_1'][-1])),
            (f2, 10, 'extra_conv_2', 'pred_2', None),
            (f3, 5, 'extra_conv_3', 'pred_3', None)):
        we, be = _w3(params[ek])
        h = _conv3(f, we, be, (sdim, sdim, sdim), pre=pre)
        wp = params[pk]['w']
        wp = wp.reshape(wp.shape[0], wp.shape[1]).T
        preds.append(_mm(h, wp, params[pk]['b'], act=False))

    outs = [_decode(preds[i], (20, 10, 5)[i], float(_STRIDES[i]), _ANCH[i])
            for i in range(3)]
    boxes = jnp.concatenate([o[0] for o in outs], 1)
    scores = jnp.concatenate([o[1] for o in outs], 1)
    return boxes, scores
